# baseline (speedup 1.0000x reference)
import functools

import jax
import jax.numpy as jnp
from jax import lax
from jax.experimental import pallas as pl
from jax.experimental.pallas import tpu as pltpu
from jax.experimental.pallas import tpu_sc as plsc


def _scale_kernel(kr_ref, x_ref, out_ref):
    out_ref[...] = x_ref[...] * kr_ref[0]


_LANES = 16
_NW = 32
_BUF = 27648


def _mask_fill_body(kr_hbm, mask_hbm, kr_v, buf_v, sem):
    nc = lax.axis_size("c")
    wid = lax.axis_index("s") * nc + lax.axis_index("c")
    per_w = mask_hbm.shape[0] // _NW
    n_dma = per_w // _BUF

    pltpu.sync_copy(kr_hbm, kr_v)
    krv = kr_v[...]

    def fill(i, carry):
        b = i * (4 * _LANES)
        buf_v[pl.ds(b, _LANES)] = krv
        buf_v[pl.ds(b + _LANES, _LANES)] = krv
        buf_v[pl.ds(b + 2 * _LANES, _LANES)] = krv
        buf_v[pl.ds(b + 3 * _LANES, _LANES)] = krv
        return carry

    lax.fori_loop(0, _BUF // (4 * _LANES), fill, 0)

    base = wid * per_w
    copies = [
        pltpu.async_copy(buf_v, mask_hbm.at[pl.ds(base + j * _BUF, _BUF)], sem)
        for j in range(n_dma)
    ]
    for c in copies:
        c.wait()


def kernel(x, W1, b1, W2, b2, keep_ratio):
    Bs, Ns, Cs = x.shape
    rows = Bs * Ns
    xf = x.reshape(rows, Cs)
    kr = jnp.asarray(keep_ratio, x.dtype)

    tile = 3072
    if rows % tile != 0:
        tile = 512 if rows % 512 == 0 else 8
    grid = rows // tile

    out = pl.pallas_call(
        _scale_kernel,
        grid=(grid,),
        in_specs=[
            pl.BlockSpec(memory_space=pltpu.SMEM),
            pl.BlockSpec((tile, Cs), lambda i: (i, 0)),
        ],
        out_specs=pl.BlockSpec((tile, Cs), lambda i: (i, 0)),
        out_shape=jax.ShapeDtypeStruct((rows, Cs), x.dtype),
        compiler_params=pltpu.CompilerParams(
            dimension_semantics=("parallel",),
        ),
    )(kr.reshape(1), xf)

    total = rows * Cs
    if total % (_NW * _BUF) == 0:
        mesh = plsc.VectorSubcoreMesh(core_axis_name="c", subcore_axis_name="s")
        fill = pl.kernel(
            _mask_fill_body,
            out_type=jax.ShapeDtypeStruct((total,), x.dtype),
            mesh=mesh,
            scratch_types=[
                pltpu.VMEM((_LANES,), x.dtype),
                pltpu.VMEM((_BUF,), x.dtype),
                pltpu.SemaphoreType.DMA,
            ],
        )
        mask = fill(jnp.full((_LANES,), kr, x.dtype)).reshape(rows, Cs)
    else:
        mask = pl.pallas_call(
            lambda kr_ref, m_ref: m_ref.__setitem__(
                ..., jnp.full(m_ref.shape, kr_ref[0], m_ref.dtype)
            ),
            grid=(grid,),
            in_specs=[pl.BlockSpec(memory_space=pltpu.SMEM)],
            out_specs=pl.BlockSpec((tile, Cs), lambda i: (i, 0)),
            out_shape=jax.ShapeDtypeStruct((rows, Cs), x.dtype),
            compiler_params=pltpu.CompilerParams(
                dimension_semantics=("parallel",),
            ),
        )(kr.reshape(1))

    return out.reshape(Bs, Ns, Cs), mask.reshape(Bs, Ns, Cs)


# speedup vs baseline: 2.4794x; 2.4794x over previous
import jax
import jax.numpy as jnp
from jax.experimental import pallas as pl
from jax.experimental.pallas import tpu as pltpu


def _scale_fill_kernel(kr_ref, x_ref, out_ref, mask_ref):
    kr = kr_ref[0]
    out_ref[...] = x_ref[...] * kr
    mask_ref[...] = jnp.full(mask_ref.shape, kr, dtype=mask_ref.dtype)


def kernel(x, W1, b1, W2, b2, keep_ratio):
    Bs, Ns, Cs = x.shape
    rows = Bs * Ns
    xf = x.reshape(rows, Cs)
    kr = jnp.asarray(keep_ratio, x.dtype).reshape(1)

    tile = 3072
    if rows % tile != 0:
        tile = 512 if rows % 512 == 0 else 8
    grid = rows // tile

    out, mask = pl.pallas_call(
        _scale_fill_kernel,
        grid=(grid,),
        in_specs=[
            pl.BlockSpec(memory_space=pltpu.SMEM),
            pl.BlockSpec((tile, Cs), lambda i: (i, 0)),
        ],
        out_specs=[
            pl.BlockSpec((tile, Cs), lambda i: (i, 0)),
            pl.BlockSpec((tile, Cs), lambda i: (i, 0)),
        ],
        out_shape=[jax.ShapeDtypeStruct((rows, Cs), x.dtype)] * 2,
        compiler_params=pltpu.CompilerParams(
            dimension_semantics=("parallel",),
        ),
    )(kr, xf)
    return out.reshape(Bs, Ns, Cs), mask.reshape(Bs, Ns, Cs)
